# Initial kernel scaffold; baseline (speedup 1.0000x reference)
import jax
import jax.numpy as jnp
from jax.experimental import pallas as pl


def kernel(support_x, support_y, query_x, query_y, params):
    raise NotImplementedError("write your pallas kernel here")


# speedup vs baseline: 1.8306x; 1.8306x over previous
import functools

import jax
import jax.numpy as jnp
from jax import lax
from jax.experimental import pallas as pl
from jax.experimental.pallas import tpu as pltpu

N_WAY = 2
K_SHOT = 1
IN_CH = 9
NPTS = 2048
KNN_K = 20
OUT_DIM = 64
EPS_BN = 1e-5
NEG_SLOPE = 0.2

B_ALL = 4
N = NPTS
R_KA = 256
R_KB = 512


def _dot(a, b, dims, precision=None):
    return lax.dot_general(a, b, (dims, ((), ())), precision=precision,
                           preferred_element_type=jnp.float32)


def _lrelu(x):
    return jnp.where(x >= 0, x, NEG_SLOPE * x)


def _ka_body(C, x_nc_ref, x_cn_ref, xr_ref, w1_ref, e1_ref, stats_ref, xx_s):
    b = pl.program_id(0)
    r = pl.program_id(1)

    @pl.when(r == 0)
    def _():
        x_cn = x_cn_ref[0]
        xx_s[...] = jnp.sum(x_cn * x_cn, axis=0, keepdims=True)

    xr = xr_ref[0]
    gram = _dot(xr, x_cn_ref[0], ((1,), (0,)))
    xxr = jnp.sum(xr * xr, axis=1, keepdims=True)
    inner = -2.0 * gram
    d = -xxr - inner - xx_s[...]

    iot = lax.broadcasted_iota(jnp.int32, (R_KA, N), 1)
    w1 = w1_ref[...]
    a_w = w1[:, :C]
    b_w = w1[:, C:]
    central = _dot(xr, b_w, ((1,), (1,)))

    s_acc = jnp.zeros((OUT_DIM,), jnp.float32)
    ss_acc = jnp.zeros((OUT_DIM,), jnp.float32)
    for t in range(KNN_K):
        m = jnp.max(d, axis=1, keepdims=True)
        cand = jnp.where(d == m, iot, N)
        amin = jnp.min(cand, axis=1, keepdims=True)
        oh = iot == amin
        ohf = oh.astype(jnp.float32)
        xg = _dot(ohf, x_nc_ref[0], ((1,), (0,)),
                  precision=lax.Precision.HIGHEST)
        e1_t = _dot(xg - xr, a_w, ((1,), (1,))) + central
        e1_ref[0, t] = e1_t
        s_acc = s_acc + jnp.sum(e1_t, axis=0)
        ss_acc = ss_acc + jnp.sum(e1_t * e1_t, axis=0)
        d = jnp.where(oh, -jnp.inf, d)

    @pl.when(jnp.logical_and(b % 2 == 0, r == 0))
    def _():
        stats_ref[...] = jnp.zeros(stats_ref.shape, jnp.float32)

    stats_ref[0, 0, :] += s_acc
    stats_ref[0, 1, :] += ss_acc


def _run_ka(x_nc, x_cn, w1):
    C = x_nc.shape[-1]
    grid = (B_ALL, N // R_KA)
    return pl.pallas_call(
        functools.partial(_ka_body, C),
        grid=grid,
        in_specs=[
            pl.BlockSpec((1, N, C), lambda b, r: (b, 0, 0)),
            pl.BlockSpec((1, C, N), lambda b, r: (b, 0, 0)),
            pl.BlockSpec((1, R_KA, C), lambda b, r: (b, r, 0)),
            pl.BlockSpec((OUT_DIM, 2 * C), lambda b, r: (0, 0)),
        ],
        out_specs=[
            pl.BlockSpec((1, KNN_K, R_KA, OUT_DIM), lambda b, r: (b, 0, r, 0)),
            pl.BlockSpec((1, 2, OUT_DIM), lambda b, r: (b // 2, 0, 0)),
        ],
        out_shape=[
            jax.ShapeDtypeStruct((B_ALL, KNN_K, N, OUT_DIM), jnp.float32),
            jax.ShapeDtypeStruct((2, 2, OUT_DIM), jnp.float32),
        ],
        scratch_shapes=[
            pltpu.VMEM((1, N), jnp.float32),
        ],
    )(x_nc, x_cn, x_nc, w1)


def _kb_body(e1_ref, stats_ref, w2_ref, g1_ref, b1_ref, ymax_ref, szg_ref):
    b = pl.program_id(0)
    r = pl.program_id(1)
    m_cnt = 2.0 * N * KNN_K
    s1 = stats_ref[0, 0, :]
    ss1 = stats_ref[0, 1, :]
    mean1 = s1 / m_cnt
    var1 = ss1 / m_cnt - mean1 * mean1
    scale = g1_ref[0] * lax.rsqrt(var1 + EPS_BN)
    shift = b1_ref[0] - mean1 * scale

    e1 = jnp.reshape(e1_ref[0], (KNN_K * R_KB, OUT_DIM))
    z = _lrelu(e1 * scale + shift)
    y = _dot(z, w2_ref[...], ((1,), (1,)))
    ymax_ref[0] = jnp.max(jnp.reshape(y, (KNN_K, R_KB, OUT_DIM)), axis=0)

    @pl.when(jnp.logical_and(b % 2 == 0, r == 0))
    def _():
        szg_ref[...] = jnp.zeros(szg_ref.shape, jnp.float32)

    szg_ref[0, 0, :] += jnp.sum(z, axis=0)
    szg_ref[0, 1:, :] += _dot(z, z, ((0,), (0,)),
                              precision=lax.Precision.HIGHEST)


def _run_kb(e1, stats1, w2, g1, b1):
    grid = (B_ALL, N // R_KB)
    return pl.pallas_call(
        _kb_body,
        grid=grid,
        in_specs=[
            pl.BlockSpec((1, KNN_K, R_KB, OUT_DIM), lambda b, r: (b, 0, r, 0)),
            pl.BlockSpec((1, 2, OUT_DIM), lambda b, r: (b // 2, 0, 0)),
            pl.BlockSpec((OUT_DIM, OUT_DIM), lambda b, r: (0, 0)),
            pl.BlockSpec((1, OUT_DIM), lambda b, r: (0, 0)),
            pl.BlockSpec((1, OUT_DIM), lambda b, r: (0, 0)),
        ],
        out_specs=[
            pl.BlockSpec((1, R_KB, OUT_DIM), lambda b, r: (b, r, 0)),
            pl.BlockSpec((1, 1 + OUT_DIM, OUT_DIM), lambda b, r: (b // 2, 0, 0)),
        ],
        out_shape=[
            jax.ShapeDtypeStruct((B_ALL, N, OUT_DIM), jnp.float32),
            jax.ShapeDtypeStruct((2, 1 + OUT_DIM, OUT_DIM), jnp.float32),
        ],
    )(e1, stats1, w2, g1.reshape(1, -1), b1.reshape(1, -1))


def _kc_body(ymax_ref, szg_ref, w2_ref, g2_ref, b2_ref, xout_ref, xout_t_ref):
    m_cnt = 2.0 * N * KNN_K
    w2 = w2_ref[...]
    for g in range(2):
        s = szg_ref[g, 0:1, :]
        gm = szg_ref[g, 1:, :]
        mean2 = _dot(s, w2, ((1,), (1,)),
                     precision=lax.Precision.HIGHEST) / m_cnt
        t = _dot(w2, gm, ((1,), (0,)), precision=lax.Precision.HIGHEST)
        e2 = jnp.sum(t * w2, axis=1, keepdims=True).T / m_cnt
        var2 = e2 - mean2 * mean2
        scale = g2_ref[0] * lax.rsqrt(var2 + EPS_BN)
        shift = b2_ref[0] - mean2 * scale
        for bb in range(2):
            cloud = 2 * g + bb
            xo = _lrelu(ymax_ref[cloud] * scale + shift)
            xout_ref[cloud] = xo
            xout_t_ref[cloud] = xo.T


def _run_kc(ymax, szg, w2, g2, b2):
    return pl.pallas_call(
        _kc_body,
        in_specs=[
            pl.BlockSpec((B_ALL, N, OUT_DIM), lambda: (0, 0, 0)),
            pl.BlockSpec((2, 1 + OUT_DIM, OUT_DIM), lambda: (0, 0, 0)),
            pl.BlockSpec((OUT_DIM, OUT_DIM), lambda: (0, 0)),
            pl.BlockSpec((1, OUT_DIM), lambda: (0, 0)),
            pl.BlockSpec((1, OUT_DIM), lambda: (0, 0)),
        ],
        out_specs=[
            pl.BlockSpec((B_ALL, N, OUT_DIM), lambda: (0, 0, 0)),
            pl.BlockSpec((B_ALL, OUT_DIM, N), lambda: (0, 0, 0)),
        ],
        out_shape=[
            jax.ShapeDtypeStruct((B_ALL, N, OUT_DIM), jnp.float32),
            jax.ShapeDtypeStruct((B_ALL, OUT_DIM, N), jnp.float32),
        ],
    )(ymax, szg, w2, g2.reshape(1, -1), b2.reshape(1, -1))


def _lin_body(has_stats, act, f_ref, stats_in_ref, w_ref, bias_ref,
              g_ref, bsh_ref, y_ref, stats_ref):
    b = pl.program_id(0)
    f = f_ref[0]
    if has_stats:
        m_cnt = 2.0 * N
        s = stats_in_ref[0, 0, :]
        ss = stats_in_ref[0, 1, :]
        mean = s / m_cnt
        var = ss / m_cnt - mean * mean
        scale = g_ref[0] * lax.rsqrt(var + EPS_BN)
        shift = bsh_ref[0] - mean * scale
        f = f * scale + shift
        if act == "lrelu":
            f = _lrelu(f)
        elif act == "relu":
            f = jnp.maximum(f, 0.0)
    y = _dot(f, w_ref[...], ((1,), (1,)))
    if bias_ref is not None:
        y = y + bias_ref[0]
    y_ref[0] = y

    @pl.when(b % 2 == 0)
    def _():
        stats_ref[...] = jnp.zeros(stats_ref.shape, jnp.float32)

    stats_ref[0, 0, :] += jnp.sum(y, axis=0)
    stats_ref[0, 1, :] += jnp.sum(y * y, axis=0)


def _run_lin(f, w, bias=None, stats_in=None, g=None, bsh=None, act="lrelu"):
    cin = f.shape[-1]
    cout = w.shape[0]
    has_stats = stats_in is not None
    in_specs = [pl.BlockSpec((1, N, cin), lambda b: (b, 0, 0))]
    args = [f]
    if has_stats:
        in_specs.append(pl.BlockSpec((1, 2, cin), lambda b: (b // 2, 0, 0)))
        args.append(stats_in)
    in_specs.append(pl.BlockSpec((cout, cin), lambda b: (0, 0)))
    args.append(w)
    if bias is not None:
        in_specs.append(pl.BlockSpec((1, cout), lambda b: (0, 0)))
        args.append(bias.reshape(1, -1))
    if has_stats:
        in_specs.append(pl.BlockSpec((1, cin), lambda b: (0, 0)))
        args.append(g.reshape(1, -1))
        in_specs.append(pl.BlockSpec((1, cin), lambda b: (0, 0)))
        args.append(bsh.reshape(1, -1))

    def wrapped(*refs):
        if has_stats:
            if bias is not None:
                f_r, si_r, w_r, bias_r, g_r, bsh_r, y_r, st_r = refs
            else:
                f_r, si_r, w_r, g_r, bsh_r, y_r, st_r = refs
                bias_r = None
            _lin_body(True, act, f_r, si_r, w_r, bias_r, g_r, bsh_r, y_r, st_r)
        else:
            if bias is not None:
                f_r, w_r, bias_r, y_r, st_r = refs
            else:
                f_r, w_r, y_r, st_r = refs
                bias_r = None
            _lin_body(False, act, f_r, None, w_r, bias_r, None, None, y_r, st_r)

    return pl.pallas_call(
        wrapped,
        grid=(B_ALL,),
        in_specs=in_specs,
        out_specs=[
            pl.BlockSpec((1, N, cout), lambda b: (b, 0, 0)),
            pl.BlockSpec((1, 2, cout), lambda b: (b // 2, 0, 0)),
        ],
        out_shape=[
            jax.ShapeDtypeStruct((B_ALL, N, cout), jnp.float32),
            jax.ShapeDtypeStruct((2, 2, cout), jnp.float32),
        ],
    )(*args)


def _bnact_body(act, y_ref, stats_ref, g_ref, bsh_ref, out_ref):
    m_cnt = 2.0 * N
    s = stats_ref[0, 0, :]
    ss = stats_ref[0, 1, :]
    mean = s / m_cnt
    var = ss / m_cnt - mean * mean
    scale = g_ref[0] * lax.rsqrt(var + EPS_BN)
    shift = bsh_ref[0] - mean * scale
    y = y_ref[0] * scale + shift
    if act == "lrelu":
        y = _lrelu(y)
    elif act == "relu":
        y = jnp.maximum(y, 0.0)
    out_ref[0] = y


def _run_bnact(y, stats, g, bsh, act="lrelu"):
    c = y.shape[-1]
    return pl.pallas_call(
        functools.partial(_bnact_body, act),
        grid=(B_ALL,),
        in_specs=[
            pl.BlockSpec((1, N, c), lambda b: (b, 0, 0)),
            pl.BlockSpec((1, 2, c), lambda b: (b // 2, 0, 0)),
            pl.BlockSpec((1, c), lambda b: (0, 0)),
            pl.BlockSpec((1, c), lambda b: (0, 0)),
        ],
        out_specs=pl.BlockSpec((1, N, c), lambda b: (b, 0, 0)),
        out_shape=jax.ShapeDtypeStruct((B_ALL, N, c), jnp.float32),
    )(y, stats, g.reshape(1, -1), bsh.reshape(1, -1))


def _attn_body(f_ref, wq_ref, wk_ref, wv_ref, out_ref):
    f = f_ref[0]
    q = _dot(f, wq_ref[...], ((1,), (1,)))
    k = _dot(f, wk_ref[...], ((1,), (1,)))
    v = _dot(f, wv_ref[...], ((1,), (1,)))
    temp = OUT_DIM ** 0.5
    logits = _dot(q / temp, k, ((1,), (1,)))
    m = jnp.max(logits, axis=1, keepdims=True)
    e = jnp.exp(logits - m)
    p = e / jnp.sum(e, axis=1, keepdims=True)
    out_ref[0] = _dot(p, v, ((1,), (0,)))


def _run_attn(f2, wq, wk, wv):
    cin = f2.shape[-1]
    return pl.pallas_call(
        _attn_body,
        grid=(B_ALL,),
        in_specs=[
            pl.BlockSpec((1, N, cin), lambda b: (b, 0, 0)),
            pl.BlockSpec((OUT_DIM, cin), lambda b: (0, 0)),
            pl.BlockSpec((OUT_DIM, cin), lambda b: (0, 0)),
            pl.BlockSpec((OUT_DIM, cin), lambda b: (0, 0)),
        ],
        out_specs=pl.BlockSpec((1, N, OUT_DIM), lambda b: (b, 0, 0)),
        out_shape=jax.ShapeDtypeStruct((B_ALL, N, OUT_DIM), jnp.float32),
    )(f2, wq, wk, wv)


def _final_body(x1_ref, att_ref, yb_ref, statsb_ref, gb_ref, bb_ref,
                sy_ref, qy_ref, pred_ref, loss_ref):
    m_cnt = 2.0 * N
    feats = []
    for g in range(2):
        s = statsb_ref[g, 0, :]
        ss = statsb_ref[g, 1, :]
        mean = s / m_cnt
        var = ss / m_cnt - mean * mean
        scale = gb_ref[0] * lax.rsqrt(var + EPS_BN)
        shift = bb_ref[0] - mean * scale
        for bb_i in range(2):
            cloud = 2 * g + bb_i
            f3 = yb_ref[cloud] * scale + shift
            feats.append(jnp.concatenate(
                [x1_ref[cloud], att_ref[cloud], f3], axis=1))

    fg_list = []
    bg_list = []
    for w in range(2):
        mask = sy_ref[w].astype(jnp.float32).reshape(N, 1)
        sf = feats[w]
        fg = jnp.sum(sf * mask, axis=0) / (jnp.sum(mask) + 1e-5)
        bgm = 1.0 - mask
        bg = jnp.sum(sf * bgm, axis=0) / (jnp.sum(bgm) + 1e-5)
        fg_list.append(fg)
        bg_list.append(bg)
    bg_proto = (bg_list[0] + bg_list[1]) / 2.0
    protos = jnp.stack([bg_proto, fg_list[0], fg_list[1]], axis=0)
    pn = jnp.maximum(jnp.sqrt(jnp.sum(protos * protos, axis=1)), 1e-8)

    loss_total = jnp.zeros((), jnp.float32)
    for b in range(2):
        qf = feats[2 + b]
        fn = jnp.maximum(jnp.sqrt(jnp.sum(qf * qf, axis=1, keepdims=True)),
                         1e-8)
        dots = _dot(qf, protos, ((1,), (1,)))
        pred = dots / (fn * pn.reshape(1, 3)) * 10.0
        pred_ref[b] = pred
        m = jnp.max(pred, axis=1, keepdims=True)
        lse = m + jnp.log(jnp.sum(jnp.exp(pred - m), axis=1, keepdims=True))
        logp = pred - lse
        qy = qy_ref[b].reshape(N, 1)
        oh = (lax.broadcasted_iota(jnp.int32, (N, 3), 1) == qy)
        loss_total = loss_total - jnp.sum(jnp.where(oh, logp, 0.0))
    loss_ref[...] = jnp.reshape(loss_total / (2.0 * N), (1, 1))


def _run_final(x1, att, yb, statsb, gb, bb, sy, qy):
    return pl.pallas_call(
        _final_body,
        in_specs=[
            pl.BlockSpec((B_ALL, N, OUT_DIM), lambda: (0, 0, 0)),
            pl.BlockSpec((B_ALL, N, OUT_DIM), lambda: (0, 0, 0)),
            pl.BlockSpec((B_ALL, N, OUT_DIM), lambda: (0, 0, 0)),
            pl.BlockSpec((2, 2, OUT_DIM), lambda: (0, 0, 0)),
            pl.BlockSpec((1, OUT_DIM), lambda: (0, 0)),
            pl.BlockSpec((1, OUT_DIM), lambda: (0, 0)),
            pl.BlockSpec((2, N), lambda: (0, 0)),
            pl.BlockSpec((2, N), lambda: (0, 0)),
        ],
        out_specs=[
            pl.BlockSpec((2, N, 3), lambda: (0, 0, 0)),
            pl.BlockSpec((1, 1), lambda: (0, 0)),
        ],
        out_shape=[
            jax.ShapeDtypeStruct((2, N, 3), jnp.float32),
            jax.ShapeDtypeStruct((1, 1), jnp.float32),
        ],
    )(x1, att, yb, statsb, gb.reshape(1, -1), bb.reshape(1, -1), sy, qy)


def kernel(support_x, support_y, query_x, query_y, params):
    p = params
    sx = support_x.reshape(N_WAY * K_SHOT, IN_CH, NPTS)
    x_cn = jnp.concatenate([sx, query_x], axis=0)
    x_nc = jnp.swapaxes(x_cn, 1, 2)

    outs = []
    for i in range(3):
        e1, stats1 = _run_ka(x_nc, x_cn, p['ec%d_0_w' % i])
        ymax, szg = _run_kb(e1, stats1, p['ec%d_1_w' % i],
                            p['ec%d_0_g' % i], p['ec%d_0_b' % i])
        x_nc, x_cn = _run_kc(ymax, szg, p['ec%d_1_w' % i],
                             p['ec%d_1_g' % i], p['ec%d_1_b' % i])
        outs.append(x_nc)

    f = jnp.concatenate(outs, axis=-1)
    y1, st1 = _run_lin(f, p['mlp0_w'])
    y2, st2 = _run_lin(y1, p['mlp1_w'], stats_in=st1,
                       g=p['mlp0_g'], bsh=p['mlp0_b'], act="lrelu")
    f2 = _run_bnact(y2, st2, p['mlp1_g'], p['mlp1_b'], act="lrelu")

    att = _run_attn(f2, p['att_q_w'], p['att_k_w'], p['att_v_w'])

    yb0, stb0 = _run_lin(f2, p['bl0_w'], bias=p['bl0_bias'])
    yb1, stb1 = _run_lin(yb0, p['bl1_w'], bias=p['bl1_bias'], stats_in=stb0,
                         g=p['bl0_g'], bsh=p['bl0_b'], act="relu")

    sy = support_y.reshape(N_WAY * K_SHOT, NPTS).astype(jnp.int32)
    qy = query_y.astype(jnp.int32)
    pred_t, loss = _run_final(outs[0], att, yb1, stb1,
                              p['bl1_g'], p['bl1_b'], sy, qy)
    pred = jnp.swapaxes(pred_t, 1, 2)
    return pred, loss.reshape(())


# speedup vs baseline: 7.8638x; 4.2958x over previous
import functools

import jax
import jax.numpy as jnp
from jax import lax
from jax.experimental import pallas as pl
from jax.experimental.pallas import tpu as pltpu
from jax.experimental.pallas import tpu_sc as plsc

N_WAY = 2
K_SHOT = 1
IN_CH = 9
NPTS = 2048
KNN_K = 20
OUT_DIM = 64
EPS_BN = 1e-5
NEG_SLOPE = 0.2

B_ALL = 4
N = NPTS
R_KA = 256
R_KB = 512


def _dot(a, b, dims, precision=None):
    return lax.dot_general(a, b, (dims, ((), ())), precision=precision,
                           preferred_element_type=jnp.float32)


def _lrelu(x):
    return jnp.where(x >= 0, x, NEG_SLOPE * x)


def _ka_body(C, x_nc_ref, x_cn_ref, xr_ref, idx_ref, xx_s):
    b = pl.program_id(0)
    r = pl.program_id(1)

    @pl.when(r == 0)
    def _():
        x_cn = x_cn_ref[0]
        xx_s[...] = jnp.sum(x_cn * x_cn, axis=0, keepdims=True)

    xr = xr_ref[0]
    gram = _dot(xr, x_cn_ref[0], ((1,), (0,)))
    xxr = jnp.sum(xr * xr, axis=1, keepdims=True)
    inner = -2.0 * gram
    d = -xxr - inner - xx_s[...]

    iot = lax.broadcasted_iota(jnp.int32, (R_KA, N), 1)
    for t in range(KNN_K):
        m = jnp.max(d, axis=1, keepdims=True)
        cand = jnp.where(d == m, iot, N)
        amin = jnp.min(cand, axis=1, keepdims=True)
        idx_ref[0, :, t:t + 1] = amin + b * N
        d = jnp.where(iot == amin, -jnp.inf, d)


def _run_ka(x_nc, x_cn):
    C = x_nc.shape[-1]
    grid = (B_ALL, N // R_KA)
    return pl.pallas_call(
        functools.partial(_ka_body, C),
        grid=grid,
        in_specs=[
            pl.BlockSpec((1, N, C), lambda b, r: (b, 0, 0)),
            pl.BlockSpec((1, C, N), lambda b, r: (b, 0, 0)),
            pl.BlockSpec((1, R_KA, C), lambda b, r: (b, r, 0)),
        ],
        out_specs=pl.BlockSpec((1, R_KA, KNN_K), lambda b, r: (b, r, 0)),
        out_shape=jax.ShapeDtypeStruct((B_ALL, N, KNN_K), jnp.int32),
        scratch_shapes=[
            pltpu.VMEM((1, N), jnp.float32),
        ],
    )(x_nc, x_cn, x_nc)


_SC_TOTAL = B_ALL * N * KNN_K
_SC_NW = 32
_SC_PER_W = _SC_TOTAL // _SC_NW
_SC_CH = 128
_SC_NBUF = 4


def _make_sc_gather(C):
    mesh = plsc.VectorSubcoreMesh(core_axis_name="c", subcore_axis_name="s")

    @functools.partial(
        pl.kernel, mesh=mesh,
        compiler_params=pltpu.CompilerParams(use_tc_tiling_on_sc=False),
        out_type=jax.ShapeDtypeStruct((_SC_TOTAL, C), jnp.float32),
        scratch_types=[
            pltpu.VMEM((_SC_PER_W,), jnp.int32),
            pltpu.VMEM((_SC_NBUF, _SC_CH, C), jnp.float32),
            pltpu.SemaphoreType.DMA,
        ],
    )
    def g(table_hbm, idx_hbm, out_hbm, idx_v, buf_v, sem):
        wid = lax.axis_index("s") * 2 + lax.axis_index("c")
        base = wid * _SC_PER_W
        pltpu.sync_copy(idx_hbm.at[pl.ds(base, _SC_PER_W)], idx_v)

        def body(j, carry):
            off = j * (_SC_NBUF * _SC_CH)
            copies = []
            for u in range(_SC_NBUF):
                cp = pltpu.async_copy(
                    table_hbm.at[idx_v.at[pl.ds(off + u * _SC_CH, _SC_CH)]],
                    buf_v.at[u], sem)
                copies.append(cp)
            for u in range(_SC_NBUF):
                copies[u].wait()
                pltpu.sync_copy(
                    buf_v.at[u],
                    out_hbm.at[pl.ds(base + off + u * _SC_CH, _SC_CH)])
            return carry

        lax.fori_loop(0, _SC_PER_W // (_SC_NBUF * _SC_CH), body, 0)

    return g


_sc_gather16 = _make_sc_gather(16)
_sc_gather64 = _make_sc_gather(OUT_DIM)


def _kb1_body(C, xg_ref, xr_ref, w1_ref, e1_ref, stats_ref):
    b = pl.program_id(0)
    r = pl.program_id(1)
    w1 = w1_ref[...]
    a_w = w1[:, :C]
    b_w = w1[:, C:]
    xr = xr_ref[0]
    xg = xg_ref[0][:, :, :C]
    edge = jnp.reshape(xg - xr[None, :, :], (KNN_K * R_KB, C))
    knn = _dot(edge, a_w, ((1,), (1,)))
    central = _dot(xr, b_w, ((1,), (1,)))
    e1 = knn + jnp.reshape(
        jnp.broadcast_to(central[None, :, :], (KNN_K, R_KB, OUT_DIM)),
        (KNN_K * R_KB, OUT_DIM))
    e1_ref[0] = jnp.reshape(e1, (KNN_K, R_KB, OUT_DIM))

    @pl.when(jnp.logical_and(b % 2 == 0, r == 0))
    def _():
        stats_ref[...] = jnp.zeros(stats_ref.shape, jnp.float32)

    stats_ref[0, 0, :] += jnp.sum(e1, axis=0)
    stats_ref[0, 1, :] += jnp.sum(e1 * e1, axis=0)


def _run_kb1(xg, x_nc, w1):
    C = x_nc.shape[-1]
    cp = xg.shape[-1]
    grid = (B_ALL, N // R_KB)
    return pl.pallas_call(
        functools.partial(_kb1_body, C),
        grid=grid,
        in_specs=[
            pl.BlockSpec((1, KNN_K, R_KB, cp), lambda b, r: (b, 0, r, 0)),
            pl.BlockSpec((1, R_KB, C), lambda b, r: (b, r, 0)),
            pl.BlockSpec((OUT_DIM, 2 * C), lambda b, r: (0, 0)),
        ],
        out_specs=[
            pl.BlockSpec((1, KNN_K, R_KB, OUT_DIM), lambda b, r: (b, 0, r, 0)),
            pl.BlockSpec((1, 2, OUT_DIM), lambda b, r: (b // 2, 0, 0)),
        ],
        out_shape=[
            jax.ShapeDtypeStruct((B_ALL, KNN_K, N, OUT_DIM), jnp.float32),
            jax.ShapeDtypeStruct((2, 2, OUT_DIM), jnp.float32),
        ],
    )(xg, x_nc, w1)


def _kb_body(e1_ref, stats_ref, w2_ref, g1_ref, b1_ref, ymax_ref, szg_ref):
    b = pl.program_id(0)
    r = pl.program_id(1)
    m_cnt = 2.0 * N * KNN_K
    s1 = stats_ref[0, 0, :]
    ss1 = stats_ref[0, 1, :]
    mean1 = s1 / m_cnt
    var1 = ss1 / m_cnt - mean1 * mean1
    scale = g1_ref[0] * lax.rsqrt(var1 + EPS_BN)
    shift = b1_ref[0] - mean1 * scale

    e1 = jnp.reshape(e1_ref[0], (KNN_K * R_KB, OUT_DIM))
    z = _lrelu(e1 * scale + shift)
    y = _dot(z, w2_ref[...], ((1,), (1,)))
    ymax_ref[0] = jnp.max(jnp.reshape(y, (KNN_K, R_KB, OUT_DIM)), axis=0)

    @pl.when(jnp.logical_and(b % 2 == 0, r == 0))
    def _():
        szg_ref[...] = jnp.zeros(szg_ref.shape, jnp.float32)

    szg_ref[0, 0, :] += jnp.sum(z, axis=0)
    szg_ref[0, 1:, :] += _dot(z, z, ((0,), (0,)),
                              precision=lax.Precision.HIGHEST)


def _run_kb(e1, stats1, w2, g1, b1):
    grid = (B_ALL, N // R_KB)
    return pl.pallas_call(
        _kb_body,
        grid=grid,
        in_specs=[
            pl.BlockSpec((1, KNN_K, R_KB, OUT_DIM), lambda b, r: (b, 0, r, 0)),
            pl.BlockSpec((1, 2, OUT_DIM), lambda b, r: (b // 2, 0, 0)),
            pl.BlockSpec((OUT_DIM, OUT_DIM), lambda b, r: (0, 0)),
            pl.BlockSpec((1, OUT_DIM), lambda b, r: (0, 0)),
            pl.BlockSpec((1, OUT_DIM), lambda b, r: (0, 0)),
        ],
        out_specs=[
            pl.BlockSpec((1, R_KB, OUT_DIM), lambda b, r: (b, r, 0)),
            pl.BlockSpec((1, 1 + OUT_DIM, OUT_DIM), lambda b, r: (b // 2, 0, 0)),
        ],
        out_shape=[
            jax.ShapeDtypeStruct((B_ALL, N, OUT_DIM), jnp.float32),
            jax.ShapeDtypeStruct((2, 1 + OUT_DIM, OUT_DIM), jnp.float32),
        ],
    )(e1, stats1, w2, g1.reshape(1, -1), b1.reshape(1, -1))


def _kc_body(ymax_ref, szg_ref, w2_ref, g2_ref, b2_ref, xout_ref, xout_t_ref):
    m_cnt = 2.0 * N * KNN_K
    w2 = w2_ref[...]
    for g in range(2):
        s = szg_ref[g, 0:1, :]
        gm = szg_ref[g, 1:, :]
        mean2 = _dot(s, w2, ((1,), (1,)),
                     precision=lax.Precision.HIGHEST) / m_cnt
        t = _dot(w2, gm, ((1,), (0,)), precision=lax.Precision.HIGHEST)
        e2 = jnp.sum(t * w2, axis=1, keepdims=True).T / m_cnt
        var2 = e2 - mean2 * mean2
        scale = g2_ref[0] * lax.rsqrt(var2 + EPS_BN)
        shift = b2_ref[0] - mean2 * scale
        for bb in range(2):
            cloud = 2 * g + bb
            xo = _lrelu(ymax_ref[cloud] * scale + shift)
            xout_ref[cloud] = xo
            xout_t_ref[cloud] = xo.T


def _run_kc(ymax, szg, w2, g2, b2):
    return pl.pallas_call(
        _kc_body,
        in_specs=[
            pl.BlockSpec((B_ALL, N, OUT_DIM), lambda: (0, 0, 0)),
            pl.BlockSpec((2, 1 + OUT_DIM, OUT_DIM), lambda: (0, 0, 0)),
            pl.BlockSpec((OUT_DIM, OUT_DIM), lambda: (0, 0)),
            pl.BlockSpec((1, OUT_DIM), lambda: (0, 0)),
            pl.BlockSpec((1, OUT_DIM), lambda: (0, 0)),
        ],
        out_specs=[
            pl.BlockSpec((B_ALL, N, OUT_DIM), lambda: (0, 0, 0)),
            pl.BlockSpec((B_ALL, OUT_DIM, N), lambda: (0, 0, 0)),
        ],
        out_shape=[
            jax.ShapeDtypeStruct((B_ALL, N, OUT_DIM), jnp.float32),
            jax.ShapeDtypeStruct((B_ALL, OUT_DIM, N), jnp.float32),
        ],
    )(ymax, szg, w2, g2.reshape(1, -1), b2.reshape(1, -1))


def _lin_body(has_stats, act, f_ref, stats_in_ref, w_ref, bias_ref,
              g_ref, bsh_ref, y_ref, stats_ref):
    b = pl.program_id(0)
    f = f_ref[0]
    if has_stats:
        m_cnt = 2.0 * N
        s = stats_in_ref[0, 0, :]
        ss = stats_in_ref[0, 1, :]
        mean = s / m_cnt
        var = ss / m_cnt - mean * mean
        scale = g_ref[0] * lax.rsqrt(var + EPS_BN)
        shift = bsh_ref[0] - mean * scale
        f = f * scale + shift
        if act == "lrelu":
            f = _lrelu(f)
        elif act == "relu":
            f = jnp.maximum(f, 0.0)
    y = _dot(f, w_ref[...], ((1,), (1,)))
    if bias_ref is not None:
        y = y + bias_ref[0]
    y_ref[0] = y

    @pl.when(b % 2 == 0)
    def _():
        stats_ref[...] = jnp.zeros(stats_ref.shape, jnp.float32)

    stats_ref[0, 0, :] += jnp.sum(y, axis=0)
    stats_ref[0, 1, :] += jnp.sum(y * y, axis=0)


def _run_lin(f, w, bias=None, stats_in=None, g=None, bsh=None, act="lrelu"):
    cin = f.shape[-1]
    cout = w.shape[0]
    has_stats = stats_in is not None
    in_specs = [pl.BlockSpec((1, N, cin), lambda b: (b, 0, 0))]
    args = [f]
    if has_stats:
        in_specs.append(pl.BlockSpec((1, 2, cin), lambda b: (b // 2, 0, 0)))
        args.append(stats_in)
    in_specs.append(pl.BlockSpec((cout, cin), lambda b: (0, 0)))
    args.append(w)
    if bias is not None:
        in_specs.append(pl.BlockSpec((1, cout), lambda b: (0, 0)))
        args.append(bias.reshape(1, -1))
    if has_stats:
        in_specs.append(pl.BlockSpec((1, cin), lambda b: (0, 0)))
        args.append(g.reshape(1, -1))
        in_specs.append(pl.BlockSpec((1, cin), lambda b: (0, 0)))
        args.append(bsh.reshape(1, -1))

    def wrapped(*refs):
        if has_stats:
            if bias is not None:
                f_r, si_r, w_r, bias_r, g_r, bsh_r, y_r, st_r = refs
            else:
                f_r, si_r, w_r, g_r, bsh_r, y_r, st_r = refs
                bias_r = None
            _lin_body(True, act, f_r, si_r, w_r, bias_r, g_r, bsh_r, y_r, st_r)
        else:
            if bias is not None:
                f_r, w_r, bias_r, y_r, st_r = refs
            else:
                f_r, w_r, y_r, st_r = refs
                bias_r = None
            _lin_body(False, act, f_r, None, w_r, bias_r, None, None, y_r, st_r)

    return pl.pallas_call(
        wrapped,
        grid=(B_ALL,),
        in_specs=in_specs,
        out_specs=[
            pl.BlockSpec((1, N, cout), lambda b: (b, 0, 0)),
            pl.BlockSpec((1, 2, cout), lambda b: (b // 2, 0, 0)),
        ],
        out_shape=[
            jax.ShapeDtypeStruct((B_ALL, N, cout), jnp.float32),
            jax.ShapeDtypeStruct((2, 2, cout), jnp.float32),
        ],
    )(*args)


def _bnact_body(act, y_ref, stats_ref, g_ref, bsh_ref, out_ref):
    m_cnt = 2.0 * N
    s = stats_ref[0, 0, :]
    ss = stats_ref[0, 1, :]
    mean = s / m_cnt
    var = ss / m_cnt - mean * mean
    scale = g_ref[0] * lax.rsqrt(var + EPS_BN)
    shift = bsh_ref[0] - mean * scale
    y = y_ref[0] * scale + shift
    if act == "lrelu":
        y = _lrelu(y)
    elif act == "relu":
        y = jnp.maximum(y, 0.0)
    out_ref[0] = y


def _run_bnact(y, stats, g, bsh, act="lrelu"):
    c = y.shape[-1]
    return pl.pallas_call(
        functools.partial(_bnact_body, act),
        grid=(B_ALL,),
        in_specs=[
            pl.BlockSpec((1, N, c), lambda b: (b, 0, 0)),
            pl.BlockSpec((1, 2, c), lambda b: (b // 2, 0, 0)),
            pl.BlockSpec((1, c), lambda b: (0, 0)),
            pl.BlockSpec((1, c), lambda b: (0, 0)),
        ],
        out_specs=pl.BlockSpec((1, N, c), lambda b: (b, 0, 0)),
        out_shape=jax.ShapeDtypeStruct((B_ALL, N, c), jnp.float32),
    )(y, stats, g.reshape(1, -1), bsh.reshape(1, -1))


def _attn_body(f_ref, wq_ref, wk_ref, wv_ref, out_ref):
    f = f_ref[0]
    q = _dot(f, wq_ref[...], ((1,), (1,)))
    k = _dot(f, wk_ref[...], ((1,), (1,)))
    v = _dot(f, wv_ref[...], ((1,), (1,)))
    temp = OUT_DIM ** 0.5
    logits = _dot(q / temp, k, ((1,), (1,)))
    m = jnp.max(logits, axis=1, keepdims=True)
    e = jnp.exp(logits - m)
    p = e / jnp.sum(e, axis=1, keepdims=True)
    out_ref[0] = _dot(p, v, ((1,), (0,)))


def _run_attn(f2, wq, wk, wv):
    cin = f2.shape[-1]
    return pl.pallas_call(
        _attn_body,
        grid=(B_ALL,),
        in_specs=[
            pl.BlockSpec((1, N, cin), lambda b: (b, 0, 0)),
            pl.BlockSpec((OUT_DIM, cin), lambda b: (0, 0)),
            pl.BlockSpec((OUT_DIM, cin), lambda b: (0, 0)),
            pl.BlockSpec((OUT_DIM, cin), lambda b: (0, 0)),
        ],
        out_specs=pl.BlockSpec((1, N, OUT_DIM), lambda b: (b, 0, 0)),
        out_shape=jax.ShapeDtypeStruct((B_ALL, N, OUT_DIM), jnp.float32),
    )(f2, wq, wk, wv)


def _final_body(x1_ref, att_ref, yb_ref, statsb_ref, gb_ref, bb_ref,
                sy_ref, qy_ref, pred_ref, loss_ref):
    m_cnt = 2.0 * N
    feats = []
    for g in range(2):
        s = statsb_ref[g, 0, :]
        ss = statsb_ref[g, 1, :]
        mean = s / m_cnt
        var = ss / m_cnt - mean * mean
        scale = gb_ref[0] * lax.rsqrt(var + EPS_BN)
        shift = bb_ref[0] - mean * scale
        for bb_i in range(2):
            cloud = 2 * g + bb_i
            f3 = yb_ref[cloud] * scale + shift
            feats.append(jnp.concatenate(
                [x1_ref[cloud], att_ref[cloud], f3], axis=1))

    fg_list = []
    bg_list = []
    for w in range(2):
        mask = sy_ref[w].astype(jnp.float32).reshape(N, 1)
        sf = feats[w]
        fg = jnp.sum(sf * mask, axis=0) / (jnp.sum(mask) + 1e-5)
        bgm = 1.0 - mask
        bg = jnp.sum(sf * bgm, axis=0) / (jnp.sum(bgm) + 1e-5)
        fg_list.append(fg)
        bg_list.append(bg)
    bg_proto = (bg_list[0] + bg_list[1]) / 2.0
    protos = jnp.stack([bg_proto, fg_list[0], fg_list[1]], axis=0)
    pn = jnp.maximum(jnp.sqrt(jnp.sum(protos * protos, axis=1)), 1e-8)

    loss_total = jnp.zeros((), jnp.float32)
    for b in range(2):
        qf = feats[2 + b]
        fn = jnp.maximum(jnp.sqrt(jnp.sum(qf * qf, axis=1, keepdims=True)),
                         1e-8)
        dots = _dot(qf, protos, ((1,), (1,)))
        pred = dots / (fn * pn.reshape(1, 3)) * 10.0
        pred_ref[b] = pred
        m = jnp.max(pred, axis=1, keepdims=True)
        lse = m + jnp.log(jnp.sum(jnp.exp(pred - m), axis=1, keepdims=True))
        logp = pred - lse
        qy = qy_ref[b].reshape(N, 1)
        oh = (lax.broadcasted_iota(jnp.int32, (N, 3), 1) == qy)
        loss_total = loss_total - jnp.sum(jnp.where(oh, logp, 0.0))
    loss_ref[...] = jnp.reshape(loss_total / (2.0 * N), (1, 1))


def _run_final(x1, att, yb, statsb, gb, bb, sy, qy):
    return pl.pallas_call(
        _final_body,
        in_specs=[
            pl.BlockSpec((B_ALL, N, OUT_DIM), lambda: (0, 0, 0)),
            pl.BlockSpec((B_ALL, N, OUT_DIM), lambda: (0, 0, 0)),
            pl.BlockSpec((B_ALL, N, OUT_DIM), lambda: (0, 0, 0)),
            pl.BlockSpec((2, 2, OUT_DIM), lambda: (0, 0, 0)),
            pl.BlockSpec((1, OUT_DIM), lambda: (0, 0)),
            pl.BlockSpec((1, OUT_DIM), lambda: (0, 0)),
            pl.BlockSpec((2, N), lambda: (0, 0)),
            pl.BlockSpec((2, N), lambda: (0, 0)),
        ],
        out_specs=[
            pl.BlockSpec((2, N, 3), lambda: (0, 0, 0)),
            pl.BlockSpec((1, 1), lambda: (0, 0)),
        ],
        out_shape=[
            jax.ShapeDtypeStruct((2, N, 3), jnp.float32),
            jax.ShapeDtypeStruct((1, 1), jnp.float32),
        ],
    )(x1, att, yb, statsb, gb.reshape(1, -1), bb.reshape(1, -1), sy, qy)


def kernel(support_x, support_y, query_x, query_y, params):
    p = params
    sx = support_x.reshape(N_WAY * K_SHOT, IN_CH, NPTS)
    x_cn = jnp.concatenate([sx, query_x], axis=0)
    x_nc = jnp.swapaxes(x_cn, 1, 2)

    outs = []
    for i in range(3):
        idx = _run_ka(x_nc, x_cn)
        idxf = jnp.reshape(jnp.transpose(idx, (0, 2, 1)), (-1,))
        if i == 0:
            table = jnp.pad(x_nc, ((0, 0), (0, 0), (0, 16 - IN_CH)))
            xg = _sc_gather16(table.reshape(B_ALL * N, 16), idxf)
            xg = xg.reshape(B_ALL, KNN_K, N, 16)
        else:
            xg = _sc_gather64(x_nc.reshape(B_ALL * N, OUT_DIM), idxf)
            xg = xg.reshape(B_ALL, KNN_K, N, OUT_DIM)
        e1, stats1 = _run_kb1(xg, x_nc, p['ec%d_0_w' % i])
        ymax, szg = _run_kb(e1, stats1, p['ec%d_1_w' % i],
                            p['ec%d_0_g' % i], p['ec%d_0_b' % i])
        x_nc, x_cn = _run_kc(ymax, szg, p['ec%d_1_w' % i],
                             p['ec%d_1_g' % i], p['ec%d_1_b' % i])
        outs.append(x_nc)

    f = jnp.concatenate(outs, axis=-1)
    y1, st1 = _run_lin(f, p['mlp0_w'])
    y2, st2 = _run_lin(y1, p['mlp1_w'], stats_in=st1,
                       g=p['mlp0_g'], bsh=p['mlp0_b'], act="lrelu")
    f2 = _run_bnact(y2, st2, p['mlp1_g'], p['mlp1_b'], act="lrelu")

    att = _run_attn(f2, p['att_q_w'], p['att_k_w'], p['att_v_w'])

    yb0, stb0 = _run_lin(f2, p['bl0_w'], bias=p['bl0_bias'])
    yb1, stb1 = _run_lin(yb0, p['bl1_w'], bias=p['bl1_bias'], stats_in=stb0,
                         g=p['bl0_g'], bsh=p['bl0_b'], act="relu")

    sy = support_y.reshape(N_WAY * K_SHOT, NPTS).astype(jnp.int32)
    qy = query_y.astype(jnp.int32)
    pred_t, loss = _run_final(outs[0], att, yb1, stb1,
                              p['bl1_g'], p['bl1_b'], sy, qy)
    pred = jnp.swapaxes(pred_t, 1, 2)
    return pred, loss.reshape(())


# speedup vs baseline: 8.5225x; 1.0838x over previous
import functools

import jax
import jax.numpy as jnp
from jax import lax
from jax.experimental import pallas as pl
from jax.experimental.pallas import tpu as pltpu
from jax.experimental.pallas import tpu_sc as plsc

N_WAY = 2
K_SHOT = 1
IN_CH = 9
NPTS = 2048
KNN_K = 20
OUT_DIM = 64
EPS_BN = 1e-5
NEG_SLOPE = 0.2

B_ALL = 4
N = NPTS
R_KA = 512
R_KB = 512


def _dot(a, b, dims, precision=None):
    return lax.dot_general(a, b, (dims, ((), ())), precision=precision,
                           preferred_element_type=jnp.float32)


def _lrelu(x):
    return jnp.where(x >= 0, x, NEG_SLOPE * x)


def _ka_body(C, x_nc_ref, x_cn_ref, xr_ref, idx_ref, xx_s):
    b = pl.program_id(0)
    r = pl.program_id(1)

    @pl.when(r == 0)
    def _():
        x_cn = x_cn_ref[0]
        xx_s[...] = jnp.sum(x_cn * x_cn, axis=0, keepdims=True)

    xr = xr_ref[0]
    gram = _dot(xr, x_cn_ref[0], ((1,), (0,)))
    xxr = jnp.sum(xr * xr, axis=1, keepdims=True)
    inner = -2.0 * gram
    d = -xxr - inner - xx_s[...]

    iot = lax.broadcasted_iota(jnp.int32, (R_KA, N), 1)
    for t in range(KNN_K):
        m = jnp.max(d, axis=1, keepdims=True)
        cand = jnp.where(d == m, iot, N)
        amin = jnp.min(cand, axis=1, keepdims=True)
        idx_ref[0, :, t:t + 1] = amin + b * N
        d = jnp.where(iot == amin, -jnp.inf, d)


def _run_ka(x_nc, x_cn):
    C = x_nc.shape[-1]
    grid = (B_ALL, N // R_KA)
    return pl.pallas_call(
        functools.partial(_ka_body, C),
        grid=grid,
        in_specs=[
            pl.BlockSpec((1, N, C), lambda b, r: (b, 0, 0)),
            pl.BlockSpec((1, C, N), lambda b, r: (b, 0, 0)),
            pl.BlockSpec((1, R_KA, C), lambda b, r: (b, r, 0)),
        ],
        out_specs=pl.BlockSpec((1, R_KA, KNN_K), lambda b, r: (b, r, 0)),
        out_shape=jax.ShapeDtypeStruct((B_ALL, N, KNN_K), jnp.int32),
        scratch_shapes=[
            pltpu.VMEM((1, N), jnp.float32),
        ],
    )(x_nc, x_cn, x_nc)


_SC_TOTAL = B_ALL * N * KNN_K
_SC_NW = 32
_SC_PER_W = _SC_TOTAL // _SC_NW
_SC_CH = 128
_SC_NBUF = 4


def _make_sc_gather(C):
    mesh = plsc.VectorSubcoreMesh(core_axis_name="c", subcore_axis_name="s")

    @functools.partial(
        pl.kernel, mesh=mesh,
        compiler_params=pltpu.CompilerParams(use_tc_tiling_on_sc=False),
        out_type=jax.ShapeDtypeStruct((_SC_TOTAL, C), jnp.float32),
        scratch_types=[
            pltpu.VMEM((_SC_PER_W,), jnp.int32),
            pltpu.VMEM((_SC_NBUF, _SC_CH, C), jnp.float32),
            pltpu.SemaphoreType.DMA,
        ],
    )
    def g(table_hbm, idx_hbm, out_hbm, idx_v, buf_v, sem):
        wid = lax.axis_index("s") * 2 + lax.axis_index("c")
        base = wid * _SC_PER_W
        pltpu.sync_copy(idx_hbm.at[pl.ds(base, _SC_PER_W)], idx_v)

        def body(j, carry):
            off = j * (_SC_NBUF * _SC_CH)
            copies = []
            for u in range(_SC_NBUF):
                cp = pltpu.async_copy(
                    table_hbm.at[idx_v.at[pl.ds(off + u * _SC_CH, _SC_CH)]],
                    buf_v.at[u], sem)
                copies.append(cp)
            for u in range(_SC_NBUF):
                copies[u].wait()
                pltpu.sync_copy(
                    buf_v.at[u],
                    out_hbm.at[pl.ds(base + off + u * _SC_CH, _SC_CH)])
            return carry

        lax.fori_loop(0, _SC_PER_W // (_SC_NBUF * _SC_CH), body, 0)

    return g


_sc_gather16 = _make_sc_gather(16)
_sc_gather64 = _make_sc_gather(OUT_DIM)


def _edge_conv(C, xg_ref, xr_ref, w1_ref):
    w1 = w1_ref[...]
    a_w = w1[:, :C]
    b_w = w1[:, C:]
    xr = xr_ref[0]
    xg = xg_ref[0][:, :, :C]
    edge = jnp.reshape(xg - xr[None, :, :], (KNN_K * R_KB, C))
    knn = _dot(edge, a_w, ((1,), (1,)))
    central = _dot(xr, b_w, ((1,), (1,)))
    return knn + jnp.reshape(
        jnp.broadcast_to(central[None, :, :], (KNN_K, R_KB, OUT_DIM)),
        (KNN_K * R_KB, OUT_DIM))


def _kb1_body(C, xg_ref, xr_ref, w1_ref, stats_ref):
    b = pl.program_id(0)
    r = pl.program_id(1)
    e1 = _edge_conv(C, xg_ref, xr_ref, w1_ref)

    @pl.when(jnp.logical_and(b % 2 == 0, r == 0))
    def _():
        stats_ref[...] = jnp.zeros(stats_ref.shape, jnp.float32)

    stats_ref[0, 0, :] += jnp.sum(e1, axis=0)
    stats_ref[0, 1, :] += jnp.sum(e1 * e1, axis=0)


def _run_kb1(xg, x_nc, w1):
    C = x_nc.shape[-1]
    cp = xg.shape[-1]
    grid = (B_ALL, N // R_KB)
    return pl.pallas_call(
        functools.partial(_kb1_body, C),
        grid=grid,
        in_specs=[
            pl.BlockSpec((1, KNN_K, R_KB, cp), lambda b, r: (b, 0, r, 0)),
            pl.BlockSpec((1, R_KB, C), lambda b, r: (b, r, 0)),
            pl.BlockSpec((OUT_DIM, 2 * C), lambda b, r: (0, 0)),
        ],
        out_specs=pl.BlockSpec((1, 2, OUT_DIM), lambda b, r: (b // 2, 0, 0)),
        out_shape=jax.ShapeDtypeStruct((2, 2, OUT_DIM), jnp.float32),
    )(xg, x_nc, w1)


def _kb_body(C, xg_ref, xr_ref, w1_ref, stats_ref, w2_ref, g1_ref, b1_ref,
             ymax_ref, szg_ref):
    b = pl.program_id(0)
    r = pl.program_id(1)
    m_cnt = 2.0 * N * KNN_K
    s1 = stats_ref[0, 0, :]
    ss1 = stats_ref[0, 1, :]
    mean1 = s1 / m_cnt
    var1 = ss1 / m_cnt - mean1 * mean1
    scale = g1_ref[0] * lax.rsqrt(var1 + EPS_BN)
    shift = b1_ref[0] - mean1 * scale

    e1 = _edge_conv(C, xg_ref, xr_ref, w1_ref)
    z = _lrelu(e1 * scale + shift)
    y = _dot(z, w2_ref[...], ((1,), (1,)))
    ymax_ref[0] = jnp.max(jnp.reshape(y, (KNN_K, R_KB, OUT_DIM)), axis=0)

    @pl.when(jnp.logical_and(b % 2 == 0, r == 0))
    def _():
        szg_ref[...] = jnp.zeros(szg_ref.shape, jnp.float32)

    szg_ref[0, 0, :] += jnp.sum(z, axis=0)
    szg_ref[0, 1:, :] += _dot(z, z, ((0,), (0,)),
                              precision=lax.Precision.HIGHEST)


def _run_kb(xg, x_nc, w1, stats1, w2, g1, b1):
    C = x_nc.shape[-1]
    cp = xg.shape[-1]
    grid = (B_ALL, N // R_KB)
    return pl.pallas_call(
        functools.partial(_kb_body, C),
        grid=grid,
        in_specs=[
            pl.BlockSpec((1, KNN_K, R_KB, cp), lambda b, r: (b, 0, r, 0)),
            pl.BlockSpec((1, R_KB, C), lambda b, r: (b, r, 0)),
            pl.BlockSpec((OUT_DIM, 2 * C), lambda b, r: (0, 0)),
            pl.BlockSpec((1, 2, OUT_DIM), lambda b, r: (b // 2, 0, 0)),
            pl.BlockSpec((OUT_DIM, OUT_DIM), lambda b, r: (0, 0)),
            pl.BlockSpec((1, OUT_DIM), lambda b, r: (0, 0)),
            pl.BlockSpec((1, OUT_DIM), lambda b, r: (0, 0)),
        ],
        out_specs=[
            pl.BlockSpec((1, R_KB, OUT_DIM), lambda b, r: (b, r, 0)),
            pl.BlockSpec((1, 1 + OUT_DIM, OUT_DIM), lambda b, r: (b // 2, 0, 0)),
        ],
        out_shape=[
            jax.ShapeDtypeStruct((B_ALL, N, OUT_DIM), jnp.float32),
            jax.ShapeDtypeStruct((2, 1 + OUT_DIM, OUT_DIM), jnp.float32),
        ],
    )(xg, x_nc, w1, stats1, w2, g1.reshape(1, -1), b1.reshape(1, -1))


def _kc_body(ymax_ref, szg_ref, w2_ref, g2_ref, b2_ref, xout_ref, xout_t_ref):
    m_cnt = 2.0 * N * KNN_K
    w2 = w2_ref[...]
    for g in range(2):
        s = szg_ref[g, 0:1, :]
        gm = szg_ref[g, 1:, :]
        mean2 = _dot(s, w2, ((1,), (1,)),
                     precision=lax.Precision.HIGHEST) / m_cnt
        t = _dot(w2, gm, ((1,), (0,)), precision=lax.Precision.HIGHEST)
        e2 = jnp.sum(t * w2, axis=1, keepdims=True).T / m_cnt
        var2 = e2 - mean2 * mean2
        scale = g2_ref[0] * lax.rsqrt(var2 + EPS_BN)
        shift = b2_ref[0] - mean2 * scale
        for bb in range(2):
            cloud = 2 * g + bb
            xo = _lrelu(ymax_ref[cloud] * scale + shift)
            xout_ref[cloud] = xo
            xout_t_ref[cloud] = xo.T


def _run_kc(ymax, szg, w2, g2, b2):
    return pl.pallas_call(
        _kc_body,
        in_specs=[
            pl.BlockSpec((B_ALL, N, OUT_DIM), lambda: (0, 0, 0)),
            pl.BlockSpec((2, 1 + OUT_DIM, OUT_DIM), lambda: (0, 0, 0)),
            pl.BlockSpec((OUT_DIM, OUT_DIM), lambda: (0, 0)),
            pl.BlockSpec((1, OUT_DIM), lambda: (0, 0)),
            pl.BlockSpec((1, OUT_DIM), lambda: (0, 0)),
        ],
        out_specs=[
            pl.BlockSpec((B_ALL, N, OUT_DIM), lambda: (0, 0, 0)),
            pl.BlockSpec((B_ALL, OUT_DIM, N), lambda: (0, 0, 0)),
        ],
        out_shape=[
            jax.ShapeDtypeStruct((B_ALL, N, OUT_DIM), jnp.float32),
            jax.ShapeDtypeStruct((B_ALL, OUT_DIM, N), jnp.float32),
        ],
    )(ymax, szg, w2, g2.reshape(1, -1), b2.reshape(1, -1))


def _lin_body(has_stats, act, f_ref, stats_in_ref, w_ref, bias_ref,
              g_ref, bsh_ref, y_ref, stats_ref):
    b = pl.program_id(0)
    f = f_ref[0]
    if has_stats:
        m_cnt = 2.0 * N
        s = stats_in_ref[0, 0, :]
        ss = stats_in_ref[0, 1, :]
        mean = s / m_cnt
        var = ss / m_cnt - mean * mean
        scale = g_ref[0] * lax.rsqrt(var + EPS_BN)
        shift = bsh_ref[0] - mean * scale
        f = f * scale + shift
        if act == "lrelu":
            f = _lrelu(f)
        elif act == "relu":
            f = jnp.maximum(f, 0.0)
    y = _dot(f, w_ref[...], ((1,), (1,)))
    if bias_ref is not None:
        y = y + bias_ref[0]
    y_ref[0] = y

    @pl.when(b % 2 == 0)
    def _():
        stats_ref[...] = jnp.zeros(stats_ref.shape, jnp.float32)

    stats_ref[0, 0, :] += jnp.sum(y, axis=0)
    stats_ref[0, 1, :] += jnp.sum(y * y, axis=0)


def _run_lin(f, w, bias=None, stats_in=None, g=None, bsh=None, act="lrelu"):
    cin = f.shape[-1]
    cout = w.shape[0]
    has_stats = stats_in is not None
    in_specs = [pl.BlockSpec((1, N, cin), lambda b: (b, 0, 0))]
    args = [f]
    if has_stats:
        in_specs.append(pl.BlockSpec((1, 2, cin), lambda b: (b // 2, 0, 0)))
        args.append(stats_in)
    in_specs.append(pl.BlockSpec((cout, cin), lambda b: (0, 0)))
    args.append(w)
    if bias is not None:
        in_specs.append(pl.BlockSpec((1, cout), lambda b: (0, 0)))
        args.append(bias.reshape(1, -1))
    if has_stats:
        in_specs.append(pl.BlockSpec((1, cin), lambda b: (0, 0)))
        args.append(g.reshape(1, -1))
        in_specs.append(pl.BlockSpec((1, cin), lambda b: (0, 0)))
        args.append(bsh.reshape(1, -1))

    def wrapped(*refs):
        if has_stats:
            if bias is not None:
                f_r, si_r, w_r, bias_r, g_r, bsh_r, y_r, st_r = refs
            else:
                f_r, si_r, w_r, g_r, bsh_r, y_r, st_r = refs
                bias_r = None
            _lin_body(True, act, f_r, si_r, w_r, bias_r, g_r, bsh_r, y_r, st_r)
        else:
            if bias is not None:
                f_r, w_r, bias_r, y_r, st_r = refs
            else:
                f_r, w_r, y_r, st_r = refs
                bias_r = None
            _lin_body(False, act, f_r, None, w_r, bias_r, None, None, y_r, st_r)

    return pl.pallas_call(
        wrapped,
        grid=(B_ALL,),
        in_specs=in_specs,
        out_specs=[
            pl.BlockSpec((1, N, cout), lambda b: (b, 0, 0)),
            pl.BlockSpec((1, 2, cout), lambda b: (b // 2, 0, 0)),
        ],
        out_shape=[
            jax.ShapeDtypeStruct((B_ALL, N, cout), jnp.float32),
            jax.ShapeDtypeStruct((2, 2, cout), jnp.float32),
        ],
    )(*args)


def _bnact_body(act, y_ref, stats_ref, g_ref, bsh_ref, out_ref):
    m_cnt = 2.0 * N
    s = stats_ref[0, 0, :]
    ss = stats_ref[0, 1, :]
    mean = s / m_cnt
    var = ss / m_cnt - mean * mean
    scale = g_ref[0] * lax.rsqrt(var + EPS_BN)
    shift = bsh_ref[0] - mean * scale
    y = y_ref[0] * scale + shift
    if act == "lrelu":
        y = _lrelu(y)
    elif act == "relu":
        y = jnp.maximum(y, 0.0)
    out_ref[0] = y


def _run_bnact(y, stats, g, bsh, act="lrelu"):
    c = y.shape[-1]
    return pl.pallas_call(
        functools.partial(_bnact_body, act),
        grid=(B_ALL,),
        in_specs=[
            pl.BlockSpec((1, N, c), lambda b: (b, 0, 0)),
            pl.BlockSpec((1, 2, c), lambda b: (b // 2, 0, 0)),
            pl.BlockSpec((1, c), lambda b: (0, 0)),
            pl.BlockSpec((1, c), lambda b: (0, 0)),
        ],
        out_specs=pl.BlockSpec((1, N, c), lambda b: (b, 0, 0)),
        out_shape=jax.ShapeDtypeStruct((B_ALL, N, c), jnp.float32),
    )(y, stats, g.reshape(1, -1), bsh.reshape(1, -1))


def _attn_body(f_ref, wq_ref, wk_ref, wv_ref, out_ref):
    f = f_ref[0]
    q = _dot(f, wq_ref[...], ((1,), (1,)))
    k = _dot(f, wk_ref[...], ((1,), (1,)))
    v = _dot(f, wv_ref[...], ((1,), (1,)))
    temp = OUT_DIM ** 0.5
    logits = _dot(q / temp, k, ((1,), (1,)))
    m = jnp.max(logits, axis=1, keepdims=True)
    e = jnp.exp(logits - m)
    p = e / jnp.sum(e, axis=1, keepdims=True)
    out_ref[0] = _dot(p, v, ((1,), (0,)))


def _run_attn(f2, wq, wk, wv):
    cin = f2.shape[-1]
    return pl.pallas_call(
        _attn_body,
        grid=(B_ALL,),
        in_specs=[
            pl.BlockSpec((1, N, cin), lambda b: (b, 0, 0)),
            pl.BlockSpec((OUT_DIM, cin), lambda b: (0, 0)),
            pl.BlockSpec((OUT_DIM, cin), lambda b: (0, 0)),
            pl.BlockSpec((OUT_DIM, cin), lambda b: (0, 0)),
        ],
        out_specs=pl.BlockSpec((1, N, OUT_DIM), lambda b: (b, 0, 0)),
        out_shape=jax.ShapeDtypeStruct((B_ALL, N, OUT_DIM), jnp.float32),
    )(f2, wq, wk, wv)


def _final_body(x1_ref, att_ref, yb_ref, statsb_ref, gb_ref, bb_ref,
                sy_ref, qy_ref, pred_ref, loss_ref):
    m_cnt = 2.0 * N
    feats = []
    for g in range(2):
        s = statsb_ref[g, 0, :]
        ss = statsb_ref[g, 1, :]
        mean = s / m_cnt
        var = ss / m_cnt - mean * mean
        scale = gb_ref[0] * lax.rsqrt(var + EPS_BN)
        shift = bb_ref[0] - mean * scale
        for bb_i in range(2):
            cloud = 2 * g + bb_i
            f3 = yb_ref[cloud] * scale + shift
            feats.append(jnp.concatenate(
                [x1_ref[cloud], att_ref[cloud], f3], axis=1))

    fg_list = []
    bg_list = []
    for w in range(2):
        mask = sy_ref[w].astype(jnp.float32).reshape(N, 1)
        sf = feats[w]
        fg = jnp.sum(sf * mask, axis=0) / (jnp.sum(mask) + 1e-5)
        bgm = 1.0 - mask
        bg = jnp.sum(sf * bgm, axis=0) / (jnp.sum(bgm) + 1e-5)
        fg_list.append(fg)
        bg_list.append(bg)
    bg_proto = (bg_list[0] + bg_list[1]) / 2.0
    protos = jnp.stack([bg_proto, fg_list[0], fg_list[1]], axis=0)
    pn = jnp.maximum(jnp.sqrt(jnp.sum(protos * protos, axis=1)), 1e-8)

    loss_total = jnp.zeros((), jnp.float32)
    for b in range(2):
        qf = feats[2 + b]
        fn = jnp.maximum(jnp.sqrt(jnp.sum(qf * qf, axis=1, keepdims=True)),
                         1e-8)
        dots = _dot(qf, protos, ((1,), (1,)))
        pred = dots / (fn * pn.reshape(1, 3)) * 10.0
        pred_ref[b] = pred
        m = jnp.max(pred, axis=1, keepdims=True)
        lse = m + jnp.log(jnp.sum(jnp.exp(pred - m), axis=1, keepdims=True))
        logp = pred - lse
        qy = qy_ref[b].reshape(N, 1)
        oh = (lax.broadcasted_iota(jnp.int32, (N, 3), 1) == qy)
        loss_total = loss_total - jnp.sum(jnp.where(oh, logp, 0.0))
    loss_ref[...] = jnp.reshape(loss_total / (2.0 * N), (1, 1))


def _run_final(x1, att, yb, statsb, gb, bb, sy, qy):
    return pl.pallas_call(
        _final_body,
        in_specs=[
            pl.BlockSpec((B_ALL, N, OUT_DIM), lambda: (0, 0, 0)),
            pl.BlockSpec((B_ALL, N, OUT_DIM), lambda: (0, 0, 0)),
            pl.BlockSpec((B_ALL, N, OUT_DIM), lambda: (0, 0, 0)),
            pl.BlockSpec((2, 2, OUT_DIM), lambda: (0, 0, 0)),
            pl.BlockSpec((1, OUT_DIM), lambda: (0, 0)),
            pl.BlockSpec((1, OUT_DIM), lambda: (0, 0)),
            pl.BlockSpec((2, N), lambda: (0, 0)),
            pl.BlockSpec((2, N), lambda: (0, 0)),
        ],
        out_specs=[
            pl.BlockSpec((2, N, 3), lambda: (0, 0, 0)),
            pl.BlockSpec((1, 1), lambda: (0, 0)),
        ],
        out_shape=[
            jax.ShapeDtypeStruct((2, N, 3), jnp.float32),
            jax.ShapeDtypeStruct((1, 1), jnp.float32),
        ],
    )(x1, att, yb, statsb, gb.reshape(1, -1), bb.reshape(1, -1), sy, qy)


def kernel(support_x, support_y, query_x, query_y, params):
    p = params
    sx = support_x.reshape(N_WAY * K_SHOT, IN_CH, NPTS)
    x_cn = jnp.concatenate([sx, query_x], axis=0)
    x_nc = jnp.swapaxes(x_cn, 1, 2)

    outs = []
    for i in range(3):
        idx = _run_ka(x_nc, x_cn)
        idxf = jnp.reshape(jnp.transpose(idx, (0, 2, 1)), (-1,))
        if i == 0:
            table = jnp.pad(x_nc, ((0, 0), (0, 0), (0, 16 - IN_CH)))
            xg = _sc_gather16(table.reshape(B_ALL * N, 16), idxf)
            xg = xg.reshape(B_ALL, KNN_K, N, 16)
        else:
            xg = _sc_gather64(x_nc.reshape(B_ALL * N, OUT_DIM), idxf)
            xg = xg.reshape(B_ALL, KNN_K, N, OUT_DIM)
        stats1 = _run_kb1(xg, x_nc, p['ec%d_0_w' % i])
        ymax, szg = _run_kb(xg, x_nc, p['ec%d_0_w' % i], stats1,
                            p['ec%d_1_w' % i],
                            p['ec%d_0_g' % i], p['ec%d_0_b' % i])
        x_nc, x_cn = _run_kc(ymax, szg, p['ec%d_1_w' % i],
                             p['ec%d_1_g' % i], p['ec%d_1_b' % i])
        outs.append(x_nc)

    f = jnp.concatenate(outs, axis=-1)
    y1, st1 = _run_lin(f, p['mlp0_w'])
    y2, st2 = _run_lin(y1, p['mlp1_w'], stats_in=st1,
                       g=p['mlp0_g'], bsh=p['mlp0_b'], act="lrelu")
    f2 = _run_bnact(y2, st2, p['mlp1_g'], p['mlp1_b'], act="lrelu")

    att = _run_attn(f2, p['att_q_w'], p['att_k_w'], p['att_v_w'])

    yb0, stb0 = _run_lin(f2, p['bl0_w'], bias=p['bl0_bias'])
    yb1, stb1 = _run_lin(yb0, p['bl1_w'], bias=p['bl1_bias'], stats_in=stb0,
                         g=p['bl0_g'], bsh=p['bl0_b'], act="relu")

    sy = support_y.reshape(N_WAY * K_SHOT, NPTS).astype(jnp.int32)
    qy = query_y.astype(jnp.int32)
    pred_t, loss = _run_final(outs[0], att, yb1, stb1,
                              p['bl1_g'], p['bl1_b'], sy, qy)
    pred = jnp.swapaxes(pred_t, 1, 2)
    return pred, loss.reshape(())
